# baseline (speedup 1.0000x reference)
import functools

import jax
import jax.numpy as jnp
import numpy as np
from jax import lax
from jax.experimental import pallas as pl
from jax.experimental.pallas import tpu as pltpu
from jax.experimental.pallas import tpu_sc as plsc

C = 12
CP = 16
D = H = W = 160
VTOT = D * H * W
NPAD = 1 << 20

NC = 2
NS = 16
NW = NC * NS
PW = NPAD // NW
G = 128
NCH = PW // G

KROWS = VTOT * CP // 128
RW = KROWS // NS
CHR = 128
NCHT = RW // CHR

_SX = np.float32(0.5 * (W - 1))
_SY = np.float32(0.5 * (H - 1))
_SZ = np.float32(0.5 * (D - 1))


def _sc_body(xs, ys, zs, k0t, out, tbl_h, cin_v, tout_v, pts_v, idx_v,
             w_v, rows_v, out_v, tsem0, tsem1, wsem0, wsem1, psem0, psem1,
             gsem0, gsem1, osem0, osem1):
    tsem = (tsem0, tsem1)
    wsem = (wsem0, wsem1)
    psem = (psem0, psem1)
    gsem = (gsem0, gsem1)
    osem = (osem0, osem1)
    cid = lax.axis_index("c")
    sid = lax.axis_index("s")

    rbase = sid * RW

    def t_in_copies(ch, b):
        rb = rbase + ch * CHR
        return [pltpu.make_async_copy(k0t.at[pl.ds(rb, CHR)], cin_v.at[b],
                                      tsem[b])]

    def t_out_copy(ch, b):
        vb = (rbase + ch * CHR) * 8
        return pltpu.make_async_copy(tout_v.at[b],
                                     tbl_h.at[cid, pl.ds(vb, 8 * CHR)],
                                     wsem[b])

    def transpose_chunk(b):
        @pl.loop(0, CHR)
        def _row(r):
            for k in range(8):
                tout_v[b, r * 8 + k] = cin_v[b, r, pl.ds(k * CP, CP)]

    for cp in t_in_copies(0, 0):
        cp.start()

    @pl.loop(0, NCHT, step=2)
    def _tchunk(g):
        for b in (0, 1):
            ch = g + b
            nb = 1 - b

            @pl.when(ch + 1 < NCHT)
            def _fire_next():
                for cp in t_in_copies(ch + 1, nb):
                    cp.start()

            for cp in t_in_copies(ch, b):
                cp.wait()

            @pl.when(ch >= 2)
            def _reclaim():
                t_out_copy(ch - 2, b).wait()

            transpose_chunk(b)
            t_out_copy(ch, b).start()

    t_out_copy(NCHT - 2, 0).wait()
    t_out_copy(NCHT - 1, 1).wait()
    plsc.subcore_barrier()

    wid = sid * NC + cid
    wbase = wid * PW

    def pts_copies(ch, b):
        base = wbase + ch * G
        return [
            pltpu.make_async_copy(xs.at[pl.ds(base, G)], pts_v.at[b, 0], psem[b]),
            pltpu.make_async_copy(ys.at[pl.ds(base, G)], pts_v.at[b, 1], psem[b]),
            pltpu.make_async_copy(zs.at[pl.ds(base, G)], pts_v.at[b, 2], psem[b]),
        ]

    def gather_copies(b):
        return [
            pltpu.make_async_copy(tbl_h.at[cid].at[idx_v.at[b, t]],
                                  rows_v.at[b, t], gsem[b])
            for t in range(8)
        ]

    def out_copy(ch, b):
        base = (wbase + ch * G) * CP
        return pltpu.make_async_copy(out_v.at[b], out.at[pl.ds(base, G * CP)],
                                     osem[b])

    def compute_idx(b):
        for j in range(G // 16):
            sl = pl.ds(j * 16, 16)
            fx = pts_v[b, 0, sl] * _SX + _SX
            fy = pts_v[b, 1, sl] * _SY + _SY
            fz = pts_v[b, 2, sl] * _SZ + _SZ
            x0 = jnp.minimum(fx.astype(jnp.int32), W - 1)
            y0 = jnp.minimum(fy.astype(jnp.int32), H - 1)
            z0 = jnp.minimum(fz.astype(jnp.int32), D - 1)
            w_v[b, 0, sl] = fx - x0.astype(jnp.float32)
            w_v[b, 1, sl] = fy - y0.astype(jnp.float32)
            w_v[b, 2, sl] = fz - z0.astype(jnp.float32)
            x1 = jnp.minimum(x0 + 1, W - 1)
            y1 = jnp.minimum(y0 + 1, H - 1)
            z1 = jnp.minimum(z0 + 1, D - 1)
            zb0 = z0 * (H * W)
            zb1 = z1 * (H * W)
            yb0 = y0 * W
            yb1 = y1 * W
            idx_v[b, 0, sl] = zb0 + yb0 + x0
            idx_v[b, 1, sl] = zb0 + yb0 + x1
            idx_v[b, 2, sl] = zb0 + yb1 + x0
            idx_v[b, 3, sl] = zb0 + yb1 + x1
            idx_v[b, 4, sl] = zb1 + yb0 + x0
            idx_v[b, 5, sl] = zb1 + yb0 + x1
            idx_v[b, 6, sl] = zb1 + yb1 + x0
            idx_v[b, 7, sl] = zb1 + yb1 + x1

    def interp(b):
        @pl.loop(0, G // 16)
        def _grp(j):
            sl = pl.ds(j * 16, 16)
            wxv = w_v[b, 0, sl]
            wyv = w_v[b, 1, sl]
            wzv = w_v[b, 2, sl]
            for k in range(16):
                p = j * 16 + k
                wx = wxv[k]
                wy = wyv[k]
                wz = wzv[k]
                c000 = rows_v[b, 0, p]
                c001 = rows_v[b, 1, p]
                c010 = rows_v[b, 2, p]
                c011 = rows_v[b, 3, p]
                c100 = rows_v[b, 4, p]
                c101 = rows_v[b, 5, p]
                c110 = rows_v[b, 6, p]
                c111 = rows_v[b, 7, p]
                a00 = c000 + wx * (c001 - c000)
                a01 = c010 + wx * (c011 - c010)
                a10 = c100 + wx * (c101 - c100)
                a11 = c110 + wx * (c111 - c110)
                b0 = a00 + wy * (a01 - a00)
                b1 = a10 + wy * (a11 - a10)
                out_v[b, pl.ds(p * CP, CP)] = b0 + wz * (b1 - b0)

    for cp in pts_copies(0, 0):
        cp.start()
    for cp in pts_copies(1, 1):
        cp.start()
    for cp in pts_copies(0, 0):
        cp.wait()
    compute_idx(0)
    for cp in gather_copies(0):
        cp.start()

    @pl.loop(0, NCH, step=2)
    def _pair(g):
        for b in (0, 1):
            ch = g + b
            nb = 1 - b
            @pl.when(ch + 1 < NCH)
            def _stage():
                for cp in pts_copies(ch + 1, nb):
                    cp.wait()
                compute_idx(nb)
                for cp in gather_copies(nb):
                    cp.start()

            @pl.when(ch + 2 < NCH)
            def _prefetch():
                for cp in pts_copies(ch + 2, b):
                    cp.start()

            for cp in gather_copies(b):
                cp.wait()

            @pl.when(ch >= 2)
            def _reclaim():
                out_copy(ch - 2, b).wait()

            interp(b)
            out_copy(ch, b).start()

    out_copy(NCH - 2, 0).wait()
    out_copy(NCH - 1, 1).wait()


@jax.jit
def _run(xs, ys, zs, k0t):
    kern = pl.kernel(
        _sc_body,
        out_type=jax.ShapeDtypeStruct((NPAD * CP,), jnp.float32),
        mesh=plsc.VectorSubcoreMesh(core_axis_name="c", subcore_axis_name="s"),
        scratch_types=[
            pltpu.HBM((NC, VTOT, CP), jnp.float32),
            pltpu.VMEM((2, CHR, 128), jnp.float32),
            pltpu.VMEM((2, 8 * CHR, CP), jnp.float32),
            pltpu.VMEM((2, 3, G), jnp.float32),
            pltpu.VMEM((2, 8, G), jnp.int32),
            pltpu.VMEM((2, 3, G), jnp.float32),
            pltpu.VMEM((2, 8, G, CP), jnp.float32),
            pltpu.VMEM((2, G * CP), jnp.float32),
            pltpu.SemaphoreType.DMA,
            pltpu.SemaphoreType.DMA,
            pltpu.SemaphoreType.DMA,
            pltpu.SemaphoreType.DMA,
            pltpu.SemaphoreType.DMA,
            pltpu.SemaphoreType.DMA,
            pltpu.SemaphoreType.DMA,
            pltpu.SemaphoreType.DMA,
            pltpu.SemaphoreType.DMA,
            pltpu.SemaphoreType.DMA,
        ],
        compiler_params=pltpu.CompilerParams(use_tc_tiling_on_sc=False),
    )
    return kern(xs, ys, zs, k0t)


def kernel(ray_pts, k0):
    n = ray_pts.shape[0]
    k0t = jnp.pad(jnp.transpose(k0[0], (1, 2, 3, 0)),
                  ((0, 0), (0, 0), (0, 0), (0, CP - C))).reshape(KROWS, 128)
    pts = jnp.pad(ray_pts, ((0, NPAD - n), (0, 0)))
    out = _run(pts[:, 0], pts[:, 1], pts[:, 2], k0t)
    return out.reshape(NPAD, CP)[:n, :C]


# speedup vs baseline: 1.2485x; 1.2485x over previous
import functools

import jax
import jax.numpy as jnp
import numpy as np
from jax import lax
from jax.experimental import pallas as pl
from jax.experimental.pallas import tpu as pltpu
from jax.experimental.pallas import tpu_sc as plsc

C = 12
CP = 16
D = H = W = 160
VTOT = D * H * W
NPAD = 1 << 20

NC = 2
NS = 16
NW = NC * NS
PW = NPAD // NW
G = 128
NCH = PW // G

KROWS = VTOT * CP // 128
RW = KROWS // NS
CHR = 128
NCHT = RW // CHR

_SX = np.float32(0.5 * (W - 1))
_SY = np.float32(0.5 * (H - 1))
_SZ = np.float32(0.5 * (D - 1))


def _sc_body(xs, ys, zs, k0t, out, tbl_h, cin_v, tout_v, pts_v, idx_v,
             w_v, rows_v, out_v, tsem0, tsem1, wsem0, wsem1, psem0, psem1,
             gsem0, gsem1, osem0, osem1):
    tsem = (tsem0, tsem1)
    wsem = (wsem0, wsem1)
    psem = (psem0, psem1)
    gsem = (gsem0, gsem1)
    osem = (osem0, osem1)
    cid = lax.axis_index("c")
    sid = lax.axis_index("s")

    rbase = sid * RW

    def t_in_copies(ch, b):
        rb = rbase + ch * CHR
        return [pltpu.make_async_copy(k0t.at[pl.ds(rb, CHR)], cin_v.at[b],
                                      tsem[b])]

    def t_out_copy(ch, b):
        vb = (rbase + ch * CHR) * 8
        return pltpu.make_async_copy(tout_v.at[b],
                                     tbl_h.at[cid, pl.ds(vb, 8 * CHR)],
                                     wsem[b])

    def transpose_chunk(b):
        @pl.loop(0, CHR)
        def _row(r):
            for k in range(8):
                tout_v[b, r * 8 + k] = cin_v[b, r, pl.ds(k * CP, CP)]

    for cp in t_in_copies(0, 0):
        cp.start()

    @pl.loop(0, NCHT, step=2)
    def _tchunk(g):
        for b in (0, 1):
            ch = g + b
            nb = 1 - b

            @pl.when(ch + 1 < NCHT)
            def _fire_next():
                for cp in t_in_copies(ch + 1, nb):
                    cp.start()

            for cp in t_in_copies(ch, b):
                cp.wait()

            @pl.when(ch >= 2)
            def _reclaim():
                t_out_copy(ch - 2, b).wait()

            transpose_chunk(b)
            t_out_copy(ch, b).start()

    t_out_copy(NCHT - 2, 0).wait()
    t_out_copy(NCHT - 1, 1).wait()
    plsc.subcore_barrier()

    wid = sid * NC + cid
    wbase = wid * PW

    def pts_copies(ch, b):
        base = wbase + ch * G
        return [
            pltpu.make_async_copy(xs.at[pl.ds(base, G)], pts_v.at[b, 0], psem[b]),
            pltpu.make_async_copy(ys.at[pl.ds(base, G)], pts_v.at[b, 1], psem[b]),
            pltpu.make_async_copy(zs.at[pl.ds(base, G)], pts_v.at[b, 2], psem[b]),
        ]

    def gather_copies(b):
        return [
            pltpu.make_async_copy(tbl_h.at[cid].at[idx_v.at[b, t]],
                                  rows_v.at[b, t], gsem[b])
            for t in range(8)
        ]

    def out_copy(ch, b):
        base = (wbase + ch * G) * C
        return pltpu.make_async_copy(out_v.at[b, pl.ds(0, G * C)],
                                     out.at[pl.ds(base, G * C)], osem[b])

    def compute_idx(b):
        for j in range(G // 16):
            sl = pl.ds(j * 16, 16)
            fx = pts_v[b, 0, sl] * _SX + _SX
            fy = pts_v[b, 1, sl] * _SY + _SY
            fz = pts_v[b, 2, sl] * _SZ + _SZ
            x0 = jnp.minimum(fx.astype(jnp.int32), W - 1)
            y0 = jnp.minimum(fy.astype(jnp.int32), H - 1)
            z0 = jnp.minimum(fz.astype(jnp.int32), D - 1)
            w_v[b, 0, sl] = fx - x0.astype(jnp.float32)
            w_v[b, 1, sl] = fy - y0.astype(jnp.float32)
            w_v[b, 2, sl] = fz - z0.astype(jnp.float32)
            x1 = jnp.minimum(x0 + 1, W - 1)
            y1 = jnp.minimum(y0 + 1, H - 1)
            z1 = jnp.minimum(z0 + 1, D - 1)
            zb0 = z0 * (H * W)
            zb1 = z1 * (H * W)
            yb0 = y0 * W
            yb1 = y1 * W
            idx_v[b, 0, sl] = zb0 + yb0 + x0
            idx_v[b, 1, sl] = zb0 + yb0 + x1
            idx_v[b, 2, sl] = zb0 + yb1 + x0
            idx_v[b, 3, sl] = zb0 + yb1 + x1
            idx_v[b, 4, sl] = zb1 + yb0 + x0
            idx_v[b, 5, sl] = zb1 + yb0 + x1
            idx_v[b, 6, sl] = zb1 + yb1 + x0
            idx_v[b, 7, sl] = zb1 + yb1 + x1

    def interp(b):
        @pl.loop(0, G // 16)
        def _grp(j):
            sl = pl.ds(j * 16, 16)
            wxv = w_v[b, 0, sl]
            wyv = w_v[b, 1, sl]
            wzv = w_v[b, 2, sl]
            for k in range(16):
                p = j * 16 + k
                wx = wxv[k]
                wy = wyv[k]
                wz = wzv[k]
                c000 = rows_v[b, 0, p]
                c001 = rows_v[b, 1, p]
                c010 = rows_v[b, 2, p]
                c011 = rows_v[b, 3, p]
                c100 = rows_v[b, 4, p]
                c101 = rows_v[b, 5, p]
                c110 = rows_v[b, 6, p]
                c111 = rows_v[b, 7, p]
                a00 = c000 + wx * (c001 - c000)
                a01 = c010 + wx * (c011 - c010)
                a10 = c100 + wx * (c101 - c100)
                a11 = c110 + wx * (c111 - c110)
                b0 = a00 + wy * (a01 - a00)
                b1 = a10 + wy * (a11 - a10)
                out_v[b, pl.ds(p * C, CP)] = b0 + wz * (b1 - b0)

    for cp in pts_copies(0, 0):
        cp.start()
    for cp in pts_copies(1, 1):
        cp.start()
    for cp in pts_copies(0, 0):
        cp.wait()
    compute_idx(0)
    for cp in gather_copies(0):
        cp.start()

    @pl.loop(0, NCH, step=2)
    def _pair(g):
        for b in (0, 1):
            ch = g + b
            nb = 1 - b
            @pl.when(ch + 1 < NCH)
            def _stage():
                for cp in pts_copies(ch + 1, nb):
                    cp.wait()
                compute_idx(nb)
                for cp in gather_copies(nb):
                    cp.start()

            @pl.when(ch + 2 < NCH)
            def _prefetch():
                for cp in pts_copies(ch + 2, b):
                    cp.start()

            for cp in gather_copies(b):
                cp.wait()

            @pl.when(ch >= 2)
            def _reclaim():
                out_copy(ch - 2, b).wait()

            interp(b)
            out_copy(ch, b).start()

    out_copy(NCH - 2, 0).wait()
    out_copy(NCH - 1, 1).wait()


@jax.jit
def _run(xs, ys, zs, k0t):
    kern = pl.kernel(
        _sc_body,
        out_type=jax.ShapeDtypeStruct((NPAD * C,), jnp.float32),
        mesh=plsc.VectorSubcoreMesh(core_axis_name="c", subcore_axis_name="s"),
        scratch_types=[
            pltpu.HBM((NC, VTOT, CP), jnp.float32),
            pltpu.VMEM((2, CHR, 128), jnp.float32),
            pltpu.VMEM((2, 8 * CHR, CP), jnp.float32),
            pltpu.VMEM((2, 3, G), jnp.float32),
            pltpu.VMEM((2, 8, G), jnp.int32),
            pltpu.VMEM((2, 3, G), jnp.float32),
            pltpu.VMEM((2, 8, G, CP), jnp.float32),
            pltpu.VMEM((2, G * C + 4), jnp.float32),
            pltpu.SemaphoreType.DMA,
            pltpu.SemaphoreType.DMA,
            pltpu.SemaphoreType.DMA,
            pltpu.SemaphoreType.DMA,
            pltpu.SemaphoreType.DMA,
            pltpu.SemaphoreType.DMA,
            pltpu.SemaphoreType.DMA,
            pltpu.SemaphoreType.DMA,
            pltpu.SemaphoreType.DMA,
            pltpu.SemaphoreType.DMA,
        ],
        compiler_params=pltpu.CompilerParams(use_tc_tiling_on_sc=False),
    )
    return kern(xs, ys, zs, k0t)


def _tc_pack_body(in_ref, out_ref):
    a = in_ref[0, :, 0]
    eye = jnp.eye(C, CP, dtype=jnp.float32)
    vt = jax.lax.dot_general(a, eye, (((0,), (0,)), ((), ())),
                             preferred_element_type=jnp.float32)
    for q in range(W // 8):
        row = jnp.concatenate([vt[:, q * 8 + k, :] for k in range(8)], 1)
        out_ref[0, :, q, :] = row


@jax.jit
def _tc_pack(k0):
    return pl.pallas_call(
        _tc_pack_body,
        out_shape=jax.ShapeDtypeStruct((D, H, W // 8, 128), jnp.float32),
        grid=(D, H // 8),
        in_specs=[pl.BlockSpec((1, C, 1, 8, W), lambda i, j: (0, 0, i, j, 0))],
        out_specs=pl.BlockSpec((1, 8, W // 8, 128), lambda i, j: (i, j, 0, 0)),
    )(k0)


def kernel(ray_pts, k0):
    n = ray_pts.shape[0]
    k0t = _tc_pack(k0).reshape(KROWS, 128)
    pts = jnp.pad(ray_pts, ((0, NPAD - n), (0, 0)))
    out = _run(pts[:, 0], pts[:, 1], pts[:, 2], k0t)
    return out[:n * C].reshape(n, C)
